# baseline (speedup 1.0000x reference)
import functools

import jax
import jax.numpy as jnp
from jax import lax
from jax.experimental import pallas as pl
from jax.experimental.pallas import tpu as pltpu
from jax.experimental.pallas import tpu_sc as plsc

B = 16384
F = 26
D = 16
N = B * F
NC, NS, L = 2, 16, 16
NW = NC * NS
PER_W = N // NW
CHUNK = 1664
NCHUNKS = PER_W // CHUNK
FIELD_SCALE = 100000

_mesh = plsc.VectorSubcoreMesh(
    core_axis_name="c", subcore_axis_name="s", num_cores=NC, num_subcores=NS
)


@functools.partial(
    pl.kernel,
    out_type=jax.ShapeDtypeStruct((N, D), jnp.float32),
    mesh=_mesh,
    scratch_types=[
        pltpu.VMEM((CHUNK,), jnp.int32),
        pltpu.VMEM((CHUNK,), jnp.int32),
        pltpu.VMEM((CHUNK, D), jnp.float32),
        pltpu.SemaphoreType.DMA,
    ],
    compiler_params=pltpu.CompilerParams(use_tc_tiling_on_sc=False),
)
def _embed(x_hbm, table_hbm, out_hbm, idx_v, off_v, rows_v, sem):
    wid = lax.axis_index("s") * NC + lax.axis_index("c")
    base = wid * PER_W

    def fill(i, carry):
        lanes = lax.iota(jnp.int32, L) + i * L
        off_v[pl.ds(i * L, L)] = lax.rem(lanes, F) * FIELD_SCALE
        return carry

    lax.fori_loop(0, CHUNK // L, fill, 0)

    def chunk_body(c, carry):
        cb = pl.multiple_of(base + c * CHUNK, 8)
        pltpu.sync_copy(x_hbm.at[pl.ds(cb, CHUNK)], idx_v)

        def addoff(i, carry2):
            s = pl.ds(i * L, L)
            idx_v[s] = idx_v[s] + off_v[s]
            return carry2

        lax.fori_loop(0, CHUNK // L, addoff, 0)
        pltpu.async_copy(table_hbm.at[idx_v], rows_v, sem).wait()
        pltpu.sync_copy(rows_v, out_hbm.at[pl.ds(cb, CHUNK)])
        return carry

    lax.fori_loop(0, NCHUNKS, chunk_body, 0)


def kernel(x, table):
    xf = x.astype(jnp.int32).reshape(N)
    out = _embed(xf, table)
    return out.reshape(B, F, D)
